# Initial kernel scaffold; baseline (speedup 1.0000x reference)
import jax
import jax.numpy as jnp
from jax.experimental import pallas as pl


def kernel(x, edge_index, batch, edge_attr, Wn, bn_lin, We, be, eps, mlp_W, mlp_b, bn_gamma, bn_beta):
    raise NotImplementedError("write your pallas kernel here")


# speedup vs baseline: 2.8119x; 2.8119x over previous
import jax
import jax.numpy as jnp
from jax import lax
from jax.experimental import pallas as pl
from jax.experimental.pallas import tpu as pltpu
from jax.experimental.pallas import tpu_sc as plsc

N = 10000
E = 320000
D_IN = 128
D_EDGE = 16
H = 128
L = 5
ML = 4
G = 64

NC = 2
NS = 16
NW = NC * NS
EPW = E // NW
C = 80
NCHUNK = EPW // C
N_PAD = 10240
RPT = N_PAD // NS
ZR = 128


def _sc_edge_body(h_hbm, e_hbm, src_hbm, dst_hbm, agg_hbm,
                  srcv, dstv, ebuf, hbuf, zbuf, acc, sem):
    cid = lax.axis_index("c")
    sid = lax.axis_index("s")
    wid = sid * NC + cid

    def _zrow(r, carry):
        for c8 in range(H // 16):
            zbuf[r, pl.ds(c8 * 16, 16)] = jnp.zeros((16,), jnp.float32)
        return carry
    lax.fori_loop(0, ZR, _zrow, 0)
    for j in range(RPT // ZR):
        pltpu.sync_copy(zbuf, acc.at[pl.ds(sid * RPT + j * ZR, ZR)])
    plsc.subcore_barrier()

    def _chunk(k, carry):
        base = wid * EPW + k * C
        pltpu.sync_copy(src_hbm.at[pl.ds(base, C)], srcv)
        pltpu.sync_copy(dst_hbm.at[pl.ds(base, C)], dstv)
        pltpu.sync_copy(e_hbm.at[pl.ds(base, C)], ebuf)
        pltpu.async_copy(h_hbm.at[srcv], hbuf, sem).wait()

        def _row(r, c2):
            for c8 in range(H // 16):
                sl = pl.ds(c8 * 16, 16)
                ebuf[r, sl] = jnp.maximum(ebuf[r, sl] + hbuf[r, sl], 0.0)
            return c2
        lax.fori_loop(0, C, _row, 0)

        pltpu.sync_copy(ebuf, acc.at[dstv], add=True)
        return carry
    lax.fori_loop(0, NCHUNK, _chunk, 0)

    plsc.subcore_barrier()
    pltpu.sync_copy(acc.at[pl.ds(sid * RPT, RPT)],
                    agg_hbm.at[cid, pl.ds(sid * RPT, RPT)])


_sc_edge = pl.kernel(
    _sc_edge_body,
    out_type=jax.ShapeDtypeStruct((NC, N_PAD, H), jnp.float32),
    mesh=plsc.VectorSubcoreMesh(core_axis_name="c", subcore_axis_name="s"),
    scratch_types=[
        pltpu.VMEM((C,), jnp.int32),
        pltpu.VMEM((C,), jnp.int32),
        pltpu.VMEM((C, H), jnp.float32),
        pltpu.VMEM((C, H), jnp.float32),
        pltpu.VMEM((ZR, H), jnp.float32),
        pltpu.VMEM_SHARED((N_PAD, H), jnp.float32),
        pltpu.SemaphoreType.DMA,
    ],
    name="sc_gine_edge",
)


def _node_proj_body(x_ref, w_ref, b_ref, o_ref):
    o_ref[...] = (jnp.dot(x_ref[...], w_ref[...],
                          preferred_element_type=jnp.float32) + b_ref[...])


def _edge_proj_body(a_ref, w_ref, b_ref, o_ref):
    o_ref[...] = (jnp.dot(a_ref[...], w_ref[...],
                          preferred_element_type=jnp.float32) + b_ref[...])


def _layer_body(h_ref, agg_ref, w_ref, b_ref, g_ref, bt_ref, eps_ref, o_ref):
    h = h_ref[...]
    out = (1.0 + eps_ref[0]) * h + agg_ref[0] + agg_ref[1]
    for j in range(ML):
        out = jnp.dot(out, w_ref[j], preferred_element_type=jnp.float32) + b_ref[j]
        if j < ML - 1:
            out = jnp.maximum(out, 0.0)
    mu = jnp.mean(out, axis=0, keepdims=True)
    var = jnp.mean((out - mu) ** 2, axis=0, keepdims=True)
    out = g_ref[...] * (out - mu) / jnp.sqrt(var + 1e-5) + bt_ref[...]
    o_ref[...] = jnp.maximum(out, 0.0) + h


def _pool_body(h_ref, batch_ref, o_ref):
    onehot = (batch_ref[...] ==
              lax.broadcasted_iota(jnp.int32, (1, G), 1)).astype(jnp.float32)
    sums = lax.dot_general(onehot, h_ref[...], (((0,), (0,)), ((), ())),
                           preferred_element_type=jnp.float32)
    counts = lax.dot_general(onehot, jnp.ones((N, 1), jnp.float32),
                             (((0,), (0,)), ((), ())),
                             preferred_element_type=jnp.float32)
    o_ref[...] = sums / jnp.maximum(counts, 1.0)


BE = 8000


def kernel(x, edge_index, batch, edge_attr, Wn, bn_lin, We, be, eps,
           mlp_W, mlp_b, bn_gamma, bn_beta):
    src = edge_index[0].astype(jnp.int32)
    dst = edge_index[1].astype(jnp.int32)

    h = pl.pallas_call(
        _node_proj_body,
        out_shape=jax.ShapeDtypeStruct((N, H), jnp.float32),
    )(x, Wn, bn_lin.reshape(1, H))

    e = pl.pallas_call(
        _edge_proj_body,
        grid=(E // BE,),
        in_specs=[
            pl.BlockSpec((BE, D_EDGE), lambda i: (i, 0)),
            pl.BlockSpec((D_EDGE, H), lambda i: (0, 0)),
            pl.BlockSpec((1, H), lambda i: (0, 0)),
        ],
        out_specs=pl.BlockSpec((BE, H), lambda i: (i, 0)),
        out_shape=jax.ShapeDtypeStruct((E, H), jnp.float32),
    )(edge_attr, We, be.reshape(1, H))

    layer_call = pl.pallas_call(
        _layer_body,
        grid=(1,),
        in_specs=[
            pl.BlockSpec((N, H), lambda i: (0, 0)),
            pl.BlockSpec((NC, N, H), lambda i: (0, 0, 0)),
            pl.BlockSpec((ML, H, H), lambda i: (0, 0, 0)),
            pl.BlockSpec((ML, 1, H), lambda i: (0, 0, 0)),
            pl.BlockSpec((1, H), lambda i: (0, 0)),
            pl.BlockSpec((1, H), lambda i: (0, 0)),
            pl.BlockSpec(memory_space=pltpu.SMEM),
        ],
        out_specs=pl.BlockSpec((N, H), lambda i: (0, 0)),
        out_shape=jax.ShapeDtypeStruct((N, H), jnp.float32),
    )

    for i in range(L):
        agg = _sc_edge(h, e, src, dst)
        h = layer_call(h, agg, mlp_W[i], mlp_b[i].reshape(ML, 1, H),
                       bn_gamma[i].reshape(1, H), bn_beta[i].reshape(1, H),
                       eps[i].reshape(1))

    g = pl.pallas_call(
        _pool_body,
        out_shape=jax.ShapeDtypeStruct((G, H), jnp.float32),
    )(h, batch.astype(jnp.int32).reshape(N, 1))
    return g


# speedup vs baseline: 3.0364x; 1.0798x over previous
import jax
import jax.numpy as jnp
from jax import lax
from jax.experimental import pallas as pl
from jax.experimental.pallas import tpu as pltpu
from jax.experimental.pallas import tpu_sc as plsc

N = 10000
E = 320000
D_IN = 128
D_EDGE = 16
H = 128
L = 5
ML = 4
G = 64

NC = 2
NS = 16
NW = NC * NS
EPW = E // NW
C = 80
NCHUNK = EPW // C
N_PAD = 10240
RPT = N_PAD // NS
ZR = 128


def _sc_edge_body(h_hbm, e_hbm, src_hbm, dst_hbm, agg_hbm,
                  srcv, dstv, ebuf, zbuf, acc, sem):
    cid = lax.axis_index("c")
    sid = lax.axis_index("s")
    wid = sid * NC + cid

    def _zrow(r, carry):
        for c8 in range(H // 16):
            zbuf[r, pl.ds(c8 * 16, 16)] = jnp.zeros((16,), jnp.float32)
        return carry
    lax.fori_loop(0, ZR, _zrow, 0)
    for j in range(RPT // ZR):
        pltpu.sync_copy(zbuf, acc.at[pl.ds(sid * RPT + j * ZR, ZR)])
    plsc.subcore_barrier()

    def _chunk(k, carry):
        base = wid * EPW + k * C
        pltpu.sync_copy(src_hbm.at[pl.ds(base, C)], srcv)
        pltpu.sync_copy(dst_hbm.at[pl.ds(base, C)], dstv)
        pltpu.sync_copy(e_hbm.at[pl.ds(base, C)], ebuf)
        pltpu.async_copy(h_hbm.at[srcv], ebuf, sem, add=True).wait()

        @plsc.parallel_loop(0, C, step=1, unroll=4)
        def _row(r):
            for c8 in range(H // 16):
                sl = pl.ds(c8 * 16, 16)
                ebuf[r, sl] = jnp.maximum(ebuf[r, sl], 0.0)

        pltpu.sync_copy(ebuf, acc.at[dstv], add=True)
        return carry
    lax.fori_loop(0, NCHUNK, _chunk, 0)

    plsc.subcore_barrier()
    pltpu.sync_copy(acc.at[pl.ds(sid * RPT, RPT)],
                    agg_hbm.at[cid, pl.ds(sid * RPT, RPT)])


_sc_edge = pl.kernel(
    _sc_edge_body,
    out_type=jax.ShapeDtypeStruct((NC, N_PAD, H), jnp.float32),
    mesh=plsc.VectorSubcoreMesh(core_axis_name="c", subcore_axis_name="s"),
    scratch_types=[
        pltpu.VMEM((C,), jnp.int32),
        pltpu.VMEM((C,), jnp.int32),
        pltpu.VMEM((C, H), jnp.float32),
        pltpu.VMEM((ZR, H), jnp.float32),
        pltpu.VMEM_SHARED((N_PAD, H), jnp.float32),
        pltpu.SemaphoreType.DMA,
    ],
    name="sc_gine_edge",
)


def _node_proj_body(x_ref, w_ref, b_ref, o_ref):
    o_ref[...] = (jnp.dot(x_ref[...], w_ref[...],
                          preferred_element_type=jnp.float32) + b_ref[...])


def _edge_proj_body(a_ref, w_ref, b_ref, o_ref):
    o_ref[...] = (jnp.dot(a_ref[...], w_ref[...],
                          preferred_element_type=jnp.float32) + b_ref[...])


def _layer_body(h_ref, agg_ref, w_ref, b_ref, g_ref, bt_ref, eps_ref, o_ref):
    h = h_ref[...]
    out = (1.0 + eps_ref[0]) * h + agg_ref[0] + agg_ref[1]
    for j in range(ML):
        out = jnp.dot(out, w_ref[j], preferred_element_type=jnp.float32) + b_ref[j]
        if j < ML - 1:
            out = jnp.maximum(out, 0.0)
    mu = jnp.mean(out, axis=0, keepdims=True)
    var = jnp.mean((out - mu) ** 2, axis=0, keepdims=True)
    out = g_ref[...] * (out - mu) / jnp.sqrt(var + 1e-5) + bt_ref[...]
    o_ref[...] = jnp.maximum(out, 0.0) + h


def _pool_body(h_ref, batch_ref, o_ref):
    onehot = (batch_ref[...] ==
              lax.broadcasted_iota(jnp.int32, (1, G), 1)).astype(jnp.float32)
    sums = lax.dot_general(onehot, h_ref[...], (((0,), (0,)), ((), ())),
                           preferred_element_type=jnp.float32)
    counts = lax.dot_general(onehot, jnp.ones((N, 1), jnp.float32),
                             (((0,), (0,)), ((), ())),
                             preferred_element_type=jnp.float32)
    o_ref[...] = sums / jnp.maximum(counts, 1.0)


BE = 8000


def kernel(x, edge_index, batch, edge_attr, Wn, bn_lin, We, be, eps,
           mlp_W, mlp_b, bn_gamma, bn_beta):
    src = edge_index[0].astype(jnp.int32)
    dst = edge_index[1].astype(jnp.int32)

    h = pl.pallas_call(
        _node_proj_body,
        out_shape=jax.ShapeDtypeStruct((N, H), jnp.float32),
    )(x, Wn, bn_lin.reshape(1, H))

    e = pl.pallas_call(
        _edge_proj_body,
        grid=(E // BE,),
        in_specs=[
            pl.BlockSpec((BE, D_EDGE), lambda i: (i, 0)),
            pl.BlockSpec((D_EDGE, H), lambda i: (0, 0)),
            pl.BlockSpec((1, H), lambda i: (0, 0)),
        ],
        out_specs=pl.BlockSpec((BE, H), lambda i: (i, 0)),
        out_shape=jax.ShapeDtypeStruct((E, H), jnp.float32),
    )(edge_attr, We, be.reshape(1, H))

    layer_call = pl.pallas_call(
        _layer_body,
        grid=(1,),
        in_specs=[
            pl.BlockSpec((N, H), lambda i: (0, 0)),
            pl.BlockSpec((NC, N, H), lambda i: (0, 0, 0)),
            pl.BlockSpec((ML, H, H), lambda i: (0, 0, 0)),
            pl.BlockSpec((ML, 1, H), lambda i: (0, 0, 0)),
            pl.BlockSpec((1, H), lambda i: (0, 0)),
            pl.BlockSpec((1, H), lambda i: (0, 0)),
            pl.BlockSpec(memory_space=pltpu.SMEM),
        ],
        out_specs=pl.BlockSpec((N, H), lambda i: (0, 0)),
        out_shape=jax.ShapeDtypeStruct((N, H), jnp.float32),
    )

    for i in range(L):
        agg = _sc_edge(h, e, src, dst)
        h = layer_call(h, agg, mlp_W[i], mlp_b[i].reshape(ML, 1, H),
                       bn_gamma[i].reshape(1, H), bn_beta[i].reshape(1, H),
                       eps[i].reshape(1))

    g = pl.pallas_call(
        _pool_body,
        out_shape=jax.ShapeDtypeStruct((G, H), jnp.float32),
    )(h, batch.astype(jnp.int32).reshape(N, 1))
    return g


# speedup vs baseline: 4.6999x; 1.5478x over previous
import jax
import jax.numpy as jnp
from jax import lax
from jax.experimental import pallas as pl
from jax.experimental.pallas import tpu as pltpu
from jax.experimental.pallas import tpu_sc as plsc

N = 10000
E = 320000
D_IN = 128
D_EDGE = 16
H = 128
L = 5
ML = 4
G = 64

NC = 2
NS = 16
NW = NC * NS
EPW = E // NW
C = 80
NCHUNK = EPW // C
N_PAD = 10240
RPT = N_PAD // NS
ZR = 128


def _sc_edge_body(h_hbm, e_hbm, src_hbm, dst_hbm, agg_hbm,
                  srcv0, srcv1, dstv0, dstv1, ebuf0, ebuf1, zbuf, acc,
                  psem, gsem):
    cid = lax.axis_index("c")
    sid = lax.axis_index("s")
    wid = sid * NC + cid
    srcv = (srcv0, srcv1)
    dstv = (dstv0, dstv1)
    ebuf = (ebuf0, ebuf1)

    def _zrow(r, carry):
        for c8 in range(H // 16):
            zbuf[r, pl.ds(c8 * 16, 16)] = jnp.zeros((16,), jnp.float32)
        return carry
    lax.fori_loop(0, ZR, _zrow, 0)
    for j in range(RPT // ZR):
        pltpu.sync_copy(zbuf, acc.at[pl.ds(sid * RPT + j * ZR, ZR)])
    plsc.subcore_barrier()

    def _start_pre(k, b):
        base = wid * EPW + k * C
        pltpu.async_copy(src_hbm.at[pl.ds(base, C)], srcv[b], psem)
        pltpu.async_copy(dst_hbm.at[pl.ds(base, C)], dstv[b], psem)
        pltpu.async_copy(e_hbm.at[pl.ds(base, C)], ebuf[b], psem)

    def _wait_pre(k, b):
        base = wid * EPW + k * C
        pltpu.make_async_copy(src_hbm.at[pl.ds(base, C)], srcv[b], psem).wait()
        pltpu.make_async_copy(dst_hbm.at[pl.ds(base, C)], dstv[b], psem).wait()
        pltpu.make_async_copy(e_hbm.at[pl.ds(base, C)], ebuf[b], psem).wait()

    def _do_chunk(k, b, pre_k, pre_b):
        if pre_k is not None:
            _start_pre(pre_k, pre_b)
        _wait_pre(k, b)
        pltpu.async_copy(h_hbm.at[srcv[b]], ebuf[b], gsem, add=True).wait()

        @plsc.parallel_loop(0, C, step=1, unroll=4)
        def _row(r):
            for c8 in range(H // 16):
                sl = pl.ds(c8 * 16, 16)
                ebuf[b][r, sl] = jnp.maximum(ebuf[b][r, sl], 0.0)

        pltpu.sync_copy(ebuf[b], acc.at[dstv[b]], add=True)

    _start_pre(0, 0)

    def _pair(i, carry):
        k = 2 * i
        _do_chunk(k, 0, k + 1, 1)
        _do_chunk(k + 1, 1, k + 2, 0)
        return carry
    lax.fori_loop(0, NCHUNK // 2, _pair, 0)
    _do_chunk(NCHUNK - 1, 0, None, None)

    plsc.subcore_barrier()
    pltpu.sync_copy(acc.at[pl.ds(sid * RPT, RPT)],
                    agg_hbm.at[cid, pl.ds(sid * RPT, RPT)])


_sc_edge = pl.kernel(
    _sc_edge_body,
    out_type=jax.ShapeDtypeStruct((NC, N_PAD, H), jnp.float32),
    mesh=plsc.VectorSubcoreMesh(core_axis_name="c", subcore_axis_name="s"),
    scratch_types=[
        pltpu.VMEM((C,), jnp.int32),
        pltpu.VMEM((C,), jnp.int32),
        pltpu.VMEM((C,), jnp.int32),
        pltpu.VMEM((C,), jnp.int32),
        pltpu.VMEM((C, H), jnp.float32),
        pltpu.VMEM((C, H), jnp.float32),
        pltpu.VMEM((ZR, H), jnp.float32),
        pltpu.VMEM_SHARED((N_PAD, H), jnp.float32),
        pltpu.SemaphoreType.DMA,
        pltpu.SemaphoreType.DMA,
    ],
    name="sc_gine_edge",
)


def _node_proj_body(x_ref, w_ref, b_ref, o_ref):
    o_ref[...] = (jnp.dot(x_ref[...], w_ref[...],
                          preferred_element_type=jnp.float32) + b_ref[...])


def _edge_proj_body(a_ref, w_ref, b_ref, o_ref):
    o_ref[...] = (jnp.dot(a_ref[...], w_ref[...],
                          preferred_element_type=jnp.float32) + b_ref[...])


def _layer_body(h_ref, agg_ref, w_ref, b_ref, g_ref, bt_ref, eps_ref, o_ref):
    h = h_ref[...]
    out = (1.0 + eps_ref[0]) * h + agg_ref[0] + agg_ref[1]
    for j in range(ML):
        out = jnp.dot(out, w_ref[j], preferred_element_type=jnp.float32) + b_ref[j]
        if j < ML - 1:
            out = jnp.maximum(out, 0.0)
    mu = jnp.mean(out, axis=0, keepdims=True)
    var = jnp.mean((out - mu) ** 2, axis=0, keepdims=True)
    out = g_ref[...] * (out - mu) / jnp.sqrt(var + 1e-5) + bt_ref[...]
    o_ref[...] = jnp.maximum(out, 0.0) + h


def _pool_body(h_ref, batch_ref, o_ref):
    onehot = (batch_ref[...] ==
              lax.broadcasted_iota(jnp.int32, (1, G), 1)).astype(jnp.float32)
    sums = lax.dot_general(onehot, h_ref[...], (((0,), (0,)), ((), ())),
                           preferred_element_type=jnp.float32)
    counts = lax.dot_general(onehot, jnp.ones((N, 1), jnp.float32),
                             (((0,), (0,)), ((), ())),
                             preferred_element_type=jnp.float32)
    o_ref[...] = sums / jnp.maximum(counts, 1.0)


BE = 8000


def kernel(x, edge_index, batch, edge_attr, Wn, bn_lin, We, be, eps,
           mlp_W, mlp_b, bn_gamma, bn_beta):
    src = edge_index[0].astype(jnp.int32)
    dst = edge_index[1].astype(jnp.int32)

    h = pl.pallas_call(
        _node_proj_body,
        out_shape=jax.ShapeDtypeStruct((N, H), jnp.float32),
    )(x, Wn, bn_lin.reshape(1, H))

    e = pl.pallas_call(
        _edge_proj_body,
        grid=(E // BE,),
        in_specs=[
            pl.BlockSpec((BE, D_EDGE), lambda i: (i, 0)),
            pl.BlockSpec((D_EDGE, H), lambda i: (0, 0)),
            pl.BlockSpec((1, H), lambda i: (0, 0)),
        ],
        out_specs=pl.BlockSpec((BE, H), lambda i: (i, 0)),
        out_shape=jax.ShapeDtypeStruct((E, H), jnp.float32),
    )(edge_attr, We, be.reshape(1, H))

    layer_call = pl.pallas_call(
        _layer_body,
        grid=(1,),
        in_specs=[
            pl.BlockSpec((N, H), lambda i: (0, 0)),
            pl.BlockSpec((NC, N, H), lambda i: (0, 0, 0)),
            pl.BlockSpec((ML, H, H), lambda i: (0, 0, 0)),
            pl.BlockSpec((ML, 1, H), lambda i: (0, 0, 0)),
            pl.BlockSpec((1, H), lambda i: (0, 0)),
            pl.BlockSpec((1, H), lambda i: (0, 0)),
            pl.BlockSpec(memory_space=pltpu.SMEM),
        ],
        out_specs=pl.BlockSpec((N, H), lambda i: (0, 0)),
        out_shape=jax.ShapeDtypeStruct((N, H), jnp.float32),
    )

    for i in range(L):
        agg = _sc_edge(h, e, src, dst)
        h = layer_call(h, agg, mlp_W[i], mlp_b[i].reshape(ML, 1, H),
                       bn_gamma[i].reshape(1, H), bn_beta[i].reshape(1, H),
                       eps[i].reshape(1))

    g = pl.pallas_call(
        _pool_body,
        out_shape=jax.ShapeDtypeStruct((G, H), jnp.float32),
    )(h, batch.astype(jnp.int32).reshape(N, 1))
    return g


# speedup vs baseline: 5.3967x; 1.1483x over previous
import jax
import jax.numpy as jnp
from jax import lax
from jax.experimental import pallas as pl
from jax.experimental.pallas import tpu as pltpu
from jax.experimental.pallas import tpu_sc as plsc

N = 10000
E = 320000
D_IN = 128
D_EDGE = 16
H = 128
L = 5
ML = 4
G = 64

NC = 2
NS = 16
NW = NC * NS
EPW = E // NW
C = 80
NCHUNK = EPW // C
N_PAD = 10240
RPT = N_PAD // NS
ZR = 128


def _sc_edge_body(h_hbm, e_hbm, src_hbm, dst_hbm, agg_hbm,
                  srcv0, srcv1, dstv0, dstv1, ebuf0, ebuf1, zbuf, acc,
                  psem, gsem):
    cid = lax.axis_index("c")
    sid = lax.axis_index("s")
    wid = sid * NC + cid
    srcv = (srcv0, srcv1)
    dstv = (dstv0, dstv1)
    ebuf = (ebuf0, ebuf1)

    def _zrow(r, carry):
        for c8 in range(H // 16):
            zbuf[r, pl.ds(c8 * 16, 16)] = jnp.zeros((16,), jnp.float32)
        return carry
    lax.fori_loop(0, ZR, _zrow, 0)
    for j in range(RPT // ZR):
        pltpu.sync_copy(zbuf, acc.at[pl.ds(sid * RPT + j * ZR, ZR)])
    plsc.subcore_barrier()

    def _start_pre(k, b):
        base = wid * EPW + k * C
        pltpu.async_copy(src_hbm.at[pl.ds(base, C)], srcv[b], psem)
        pltpu.async_copy(dst_hbm.at[pl.ds(base, C)], dstv[b], psem)
        pltpu.async_copy(e_hbm.at[pl.ds(base, C)], ebuf[b], psem)

    def _wait_pre(k, b):
        base = wid * EPW + k * C
        pltpu.make_async_copy(src_hbm.at[pl.ds(base, C)], srcv[b], psem).wait()
        pltpu.make_async_copy(dst_hbm.at[pl.ds(base, C)], dstv[b], psem).wait()
        pltpu.make_async_copy(e_hbm.at[pl.ds(base, C)], ebuf[b], psem).wait()

    def _gather_start(b):
        pltpu.async_copy(h_hbm.at[srcv[b]], ebuf[b], gsem, add=True)

    def _gather_wait(b):
        pltpu.make_async_copy(h_hbm.at[srcv[b]], ebuf[b], gsem).wait()

    def _relu_scatter(b):
        @plsc.parallel_loop(0, C, step=1, unroll=4)
        def _row(r):
            for c8 in range(H // 16):
                sl = pl.ds(c8 * 16, 16)
                ebuf[b][r, sl] = jnp.maximum(ebuf[b][r, sl], 0.0)

        pltpu.sync_copy(ebuf[b], acc.at[dstv[b]], add=True)

    def _do_chunk(k, b, nb):
        _gather_wait(b)
        _wait_pre(k + 1, nb)
        _gather_start(nb)
        _relu_scatter(b)

        @pl.when(k + 2 < NCHUNK)
        def _():
            _start_pre(k + 2, b)

    _start_pre(0, 0)
    _wait_pre(0, 0)
    _start_pre(1, 1)
    _gather_start(0)

    def _pair(i, carry):
        k = 2 * i
        _do_chunk(k, 0, 1)
        _do_chunk(k + 1, 1, 0)
        return carry
    lax.fori_loop(0, (NCHUNK - 1) // 2, _pair, 0)
    _gather_wait(0)
    _relu_scatter(0)

    plsc.subcore_barrier()
    pltpu.sync_copy(acc.at[pl.ds(sid * RPT, RPT)],
                    agg_hbm.at[cid, pl.ds(sid * RPT, RPT)])


_sc_edge = pl.kernel(
    _sc_edge_body,
    out_type=jax.ShapeDtypeStruct((NC, N_PAD, H), jnp.float32),
    mesh=plsc.VectorSubcoreMesh(core_axis_name="c", subcore_axis_name="s"),
    scratch_types=[
        pltpu.VMEM((C,), jnp.int32),
        pltpu.VMEM((C,), jnp.int32),
        pltpu.VMEM((C,), jnp.int32),
        pltpu.VMEM((C,), jnp.int32),
        pltpu.VMEM((C, H), jnp.float32),
        pltpu.VMEM((C, H), jnp.float32),
        pltpu.VMEM((ZR, H), jnp.float32),
        pltpu.VMEM_SHARED((N_PAD, H), jnp.float32),
        pltpu.SemaphoreType.DMA,
        pltpu.SemaphoreType.DMA,
    ],
    name="sc_gine_edge",
)


def _node_proj_body(x_ref, w_ref, b_ref, o_ref):
    o_ref[...] = (jnp.dot(x_ref[...], w_ref[...],
                          preferred_element_type=jnp.float32) + b_ref[...])


def _edge_proj_body(a_ref, w_ref, b_ref, o_ref):
    o_ref[...] = (jnp.dot(a_ref[...], w_ref[...],
                          preferred_element_type=jnp.float32) + b_ref[...])


def _layer_body(h_ref, agg_ref, w_ref, b_ref, g_ref, bt_ref, eps_ref, o_ref):
    h = h_ref[...]
    out = (1.0 + eps_ref[0]) * h + agg_ref[0] + agg_ref[1]
    for j in range(ML):
        out = jnp.dot(out, w_ref[j], preferred_element_type=jnp.float32) + b_ref[j]
        if j < ML - 1:
            out = jnp.maximum(out, 0.0)
    mu = jnp.mean(out, axis=0, keepdims=True)
    var = jnp.mean((out - mu) ** 2, axis=0, keepdims=True)
    out = g_ref[...] * (out - mu) / jnp.sqrt(var + 1e-5) + bt_ref[...]
    o_ref[...] = jnp.maximum(out, 0.0) + h


def _pool_body(h_ref, batch_ref, o_ref):
    onehot = (batch_ref[...] ==
              lax.broadcasted_iota(jnp.int32, (1, G), 1)).astype(jnp.float32)
    sums = lax.dot_general(onehot, h_ref[...], (((0,), (0,)), ((), ())),
                           preferred_element_type=jnp.float32)
    counts = lax.dot_general(onehot, jnp.ones((N, 1), jnp.float32),
                             (((0,), (0,)), ((), ())),
                             preferred_element_type=jnp.float32)
    o_ref[...] = sums / jnp.maximum(counts, 1.0)


BE = 8000


def kernel(x, edge_index, batch, edge_attr, Wn, bn_lin, We, be, eps,
           mlp_W, mlp_b, bn_gamma, bn_beta):
    src = edge_index[0].astype(jnp.int32)
    dst = edge_index[1].astype(jnp.int32)

    h = pl.pallas_call(
        _node_proj_body,
        out_shape=jax.ShapeDtypeStruct((N, H), jnp.float32),
    )(x, Wn, bn_lin.reshape(1, H))

    e = pl.pallas_call(
        _edge_proj_body,
        grid=(E // BE,),
        in_specs=[
            pl.BlockSpec((BE, D_EDGE), lambda i: (i, 0)),
            pl.BlockSpec((D_EDGE, H), lambda i: (0, 0)),
            pl.BlockSpec((1, H), lambda i: (0, 0)),
        ],
        out_specs=pl.BlockSpec((BE, H), lambda i: (i, 0)),
        out_shape=jax.ShapeDtypeStruct((E, H), jnp.float32),
    )(edge_attr, We, be.reshape(1, H))

    layer_call = pl.pallas_call(
        _layer_body,
        grid=(1,),
        in_specs=[
            pl.BlockSpec((N, H), lambda i: (0, 0)),
            pl.BlockSpec((NC, N, H), lambda i: (0, 0, 0)),
            pl.BlockSpec((ML, H, H), lambda i: (0, 0, 0)),
            pl.BlockSpec((ML, 1, H), lambda i: (0, 0, 0)),
            pl.BlockSpec((1, H), lambda i: (0, 0)),
            pl.BlockSpec((1, H), lambda i: (0, 0)),
            pl.BlockSpec(memory_space=pltpu.SMEM),
        ],
        out_specs=pl.BlockSpec((N, H), lambda i: (0, 0)),
        out_shape=jax.ShapeDtypeStruct((N, H), jnp.float32),
    )

    for i in range(L):
        agg = _sc_edge(h, e, src, dst)
        h = layer_call(h, agg, mlp_W[i], mlp_b[i].reshape(ML, 1, H),
                       bn_gamma[i].reshape(1, H), bn_beta[i].reshape(1, H),
                       eps[i].reshape(1))

    g = pl.pallas_call(
        _pool_body,
        out_shape=jax.ShapeDtypeStruct((G, H), jnp.float32),
    )(h, batch.astype(jnp.int32).reshape(N, 1))
    return g


# speedup vs baseline: 6.0217x; 1.1158x over previous
import jax
import jax.numpy as jnp
from jax import lax
from jax.experimental import pallas as pl
from jax.experimental.pallas import tpu as pltpu
from jax.experimental.pallas import tpu_sc as plsc

N = 10000
E = 320000
D_IN = 128
D_EDGE = 16
H = 128
L = 5
ML = 4
G = 64

NC = 2
NS = 16
NW = NC * NS
EPW = E // NW
C = 80
NCHUNK = EPW // C
N_PAD = 10240
RPT = N_PAD // NS
ZR = 128


def _sc_edge_body(h_hbm, e_hbm, src_hbm, dst_hbm, agg_hbm,
                  srcv0, srcv1, srcv2, dstv0, dstv1, dstv2,
                  ebuf0, ebuf1, ebuf2, zbuf, acc,
                  psem, gsem, ssem0, ssem1, ssem2):
    cid = lax.axis_index("c")
    sid = lax.axis_index("s")
    wid = sid * NC + cid
    srcv = (srcv0, srcv1, srcv2)
    dstv = (dstv0, dstv1, dstv2)
    ebuf = (ebuf0, ebuf1, ebuf2)
    ssem = (ssem0, ssem1, ssem2)

    def _zrow(r, carry):
        for c8 in range(H // 16):
            zbuf[r, pl.ds(c8 * 16, 16)] = jnp.zeros((16,), jnp.float32)
        return carry
    lax.fori_loop(0, ZR, _zrow, 0)
    for j in range(RPT // ZR):
        pltpu.sync_copy(zbuf, acc.at[pl.ds(sid * RPT + j * ZR, ZR)])
    plsc.subcore_barrier()

    def _start_pre(k, b):
        base = wid * EPW + k * C
        pltpu.async_copy(src_hbm.at[pl.ds(base, C)], srcv[b], psem)
        pltpu.async_copy(dst_hbm.at[pl.ds(base, C)], dstv[b], psem)
        pltpu.async_copy(e_hbm.at[pl.ds(base, C)], ebuf[b], psem)

    def _wait_pre(k, b):
        base = wid * EPW + k * C
        pltpu.make_async_copy(src_hbm.at[pl.ds(base, C)], srcv[b], psem).wait()
        pltpu.make_async_copy(dst_hbm.at[pl.ds(base, C)], dstv[b], psem).wait()
        pltpu.make_async_copy(e_hbm.at[pl.ds(base, C)], ebuf[b], psem).wait()

    def _gather_start(b):
        pltpu.async_copy(h_hbm.at[srcv[b]], ebuf[b], gsem, add=True)

    def _gather_wait(b):
        pltpu.make_async_copy(h_hbm.at[srcv[b]], ebuf[b], gsem).wait()

    def _relu(b):
        @plsc.parallel_loop(0, C, step=1, unroll=4)
        def _row(r):
            for c8 in range(H // 16):
                sl = pl.ds(c8 * 16, 16)
                ebuf[b][r, sl] = jnp.maximum(ebuf[b][r, sl], 0.0)

    def _scatter_start(b):
        pltpu.async_copy(ebuf[b], acc.at[dstv[b]], ssem[b], add=True)

    def _scatter_wait(b):
        pltpu.make_async_copy(ebuf[b], acc.at[dstv[b]], ssem[b]).wait()

    def _do_chunk(k, b, first=False):
        nb = (b + 1) % 3
        pb = (b + 2) % 3
        _gather_wait(b)
        _wait_pre(k + 1, nb)
        _gather_start(nb)
        _relu(b)
        _scatter_start(b)
        if not first:
            _scatter_wait(pb)

        @pl.when(k + 2 < NCHUNK)
        def _():
            _start_pre(k + 2, pb)

    _start_pre(0, 0)
    _wait_pre(0, 0)
    _start_pre(1, 1)
    _gather_start(0)
    _do_chunk(0, 0, first=True)

    def _triple(i, carry):
        k = 3 * i + 1
        _do_chunk(k, 1)
        _do_chunk(k + 1, 2)
        _do_chunk(k + 2, 0)
        return carry
    lax.fori_loop(0, (NCHUNK - 1 - 1) // 3, _triple, 0)
    _gather_wait(1)
    _relu(1)
    _scatter_start(1)
    _scatter_wait(0)
    _scatter_wait(1)

    plsc.subcore_barrier()
    pltpu.sync_copy(acc.at[pl.ds(sid * RPT, RPT)],
                    agg_hbm.at[cid, pl.ds(sid * RPT, RPT)])


_sc_edge = pl.kernel(
    _sc_edge_body,
    out_type=jax.ShapeDtypeStruct((NC, N_PAD, H), jnp.float32),
    mesh=plsc.VectorSubcoreMesh(core_axis_name="c", subcore_axis_name="s"),
    scratch_types=(
        [pltpu.VMEM((C,), jnp.int32)] * 6
        + [pltpu.VMEM((C, H), jnp.float32)] * 3
        + [
            pltpu.VMEM((ZR, H), jnp.float32),
            pltpu.VMEM_SHARED((N_PAD, H), jnp.float32),
            pltpu.SemaphoreType.DMA,
            pltpu.SemaphoreType.DMA,
            pltpu.SemaphoreType.DMA,
            pltpu.SemaphoreType.DMA,
            pltpu.SemaphoreType.DMA,
        ]
    ),
    name="sc_gine_edge",
)


def _node_proj_body(x_ref, w_ref, b_ref, o_ref):
    o_ref[...] = (jnp.dot(x_ref[...], w_ref[...],
                          preferred_element_type=jnp.float32) + b_ref[...])


def _edge_proj_body(a_ref, w_ref, b_ref, o_ref):
    o_ref[...] = (jnp.dot(a_ref[...], w_ref[...],
                          preferred_element_type=jnp.float32) + b_ref[...])


def _layer_body(h_ref, agg_ref, w_ref, b_ref, g_ref, bt_ref, eps_ref, o_ref):
    h = h_ref[...]
    out = (1.0 + eps_ref[0]) * h + agg_ref[0] + agg_ref[1]
    for j in range(ML):
        out = jnp.dot(out, w_ref[j], preferred_element_type=jnp.float32) + b_ref[j]
        if j < ML - 1:
            out = jnp.maximum(out, 0.0)
    mu = jnp.mean(out, axis=0, keepdims=True)
    var = jnp.mean((out - mu) ** 2, axis=0, keepdims=True)
    out = g_ref[...] * (out - mu) / jnp.sqrt(var + 1e-5) + bt_ref[...]
    o_ref[...] = jnp.maximum(out, 0.0) + h


def _pool_body(h_ref, batch_ref, o_ref):
    onehot = (batch_ref[...] ==
              lax.broadcasted_iota(jnp.int32, (1, G), 1)).astype(jnp.float32)
    sums = lax.dot_general(onehot, h_ref[...], (((0,), (0,)), ((), ())),
                           preferred_element_type=jnp.float32)
    counts = lax.dot_general(onehot, jnp.ones((N, 1), jnp.float32),
                             (((0,), (0,)), ((), ())),
                             preferred_element_type=jnp.float32)
    o_ref[...] = sums / jnp.maximum(counts, 1.0)


BE = 8000


def kernel(x, edge_index, batch, edge_attr, Wn, bn_lin, We, be, eps,
           mlp_W, mlp_b, bn_gamma, bn_beta):
    src = edge_index[0].astype(jnp.int32)
    dst = edge_index[1].astype(jnp.int32)

    h = pl.pallas_call(
        _node_proj_body,
        out_shape=jax.ShapeDtypeStruct((N, H), jnp.float32),
    )(x, Wn, bn_lin.reshape(1, H))

    e = pl.pallas_call(
        _edge_proj_body,
        grid=(E // BE,),
        in_specs=[
            pl.BlockSpec((BE, D_EDGE), lambda i: (i, 0)),
            pl.BlockSpec((D_EDGE, H), lambda i: (0, 0)),
            pl.BlockSpec((1, H), lambda i: (0, 0)),
        ],
        out_specs=pl.BlockSpec((BE, H), lambda i: (i, 0)),
        out_shape=jax.ShapeDtypeStruct((E, H), jnp.float32),
    )(edge_attr, We, be.reshape(1, H))

    layer_call = pl.pallas_call(
        _layer_body,
        grid=(1,),
        in_specs=[
            pl.BlockSpec((N, H), lambda i: (0, 0)),
            pl.BlockSpec((NC, N, H), lambda i: (0, 0, 0)),
            pl.BlockSpec((ML, H, H), lambda i: (0, 0, 0)),
            pl.BlockSpec((ML, 1, H), lambda i: (0, 0, 0)),
            pl.BlockSpec((1, H), lambda i: (0, 0)),
            pl.BlockSpec((1, H), lambda i: (0, 0)),
            pl.BlockSpec(memory_space=pltpu.SMEM),
        ],
        out_specs=pl.BlockSpec((N, H), lambda i: (0, 0)),
        out_shape=jax.ShapeDtypeStruct((N, H), jnp.float32),
    )

    for i in range(L):
        agg = _sc_edge(h, e, src, dst)
        h = layer_call(h, agg, mlp_W[i], mlp_b[i].reshape(ML, 1, H),
                       bn_gamma[i].reshape(1, H), bn_beta[i].reshape(1, H),
                       eps[i].reshape(1))

    g = pl.pallas_call(
        _pool_body,
        out_shape=jax.ShapeDtypeStruct((G, H), jnp.float32),
    )(h, batch.astype(jnp.int32).reshape(N, 1))
    return g
